# Initial kernel scaffold; baseline (speedup 1.0000x reference)
import jax
import jax.numpy as jnp
from jax.experimental import pallas as pl


def kernel(numeric, categorical, binary, W_num, b_num, bin_emb, cat_emb_0, cat_emb_1, cat_emb_2, cat_emb_3, cat_emb_4, cat_emb_5):
    raise NotImplementedError("write your pallas kernel here")


# speedup vs baseline: 3.4995x; 3.4995x over previous
import jax
import jax.numpy as jnp
from jax import lax
from jax.experimental import pallas as pl

H = 128
NUM_F = 8
CAT_F = 6
BIN_F = 6
TOKENS = NUM_F + CAT_F + BIN_F
BLOCK_B = 512


def _tc_body(num_ref, cat_ref, bin_ref, w_ref, b_ref, be_ref,
             t0, t1, t2, t3, t4, t5, out_ref):
    tables = [t0, t1, t2, t3, t4, t5]
    for i in range(NUM_F):
        out_ref[:, i, :] = (num_ref[:, i:i + 1] * w_ref[i:i + 1, :]
                            + b_ref[i:i + 1, :])
    for i in range(CAT_F):
        vpad = tables[i].shape[0]
        idx = cat_ref[:, i:i + 1]
        oh = (idx == lax.broadcasted_iota(jnp.int32, (idx.shape[0], vpad), 1)
              ).astype(jnp.float32)
        out_ref[:, NUM_F + i, :] = jnp.dot(
            oh, tables[i][:, :], preferred_element_type=jnp.float32)
    for i in range(BIN_F):
        b = bin_ref[:, i:i + 1].astype(jnp.float32)
        e0 = be_ref[i, 0, :][None, :]
        e1 = be_ref[i, 1, :][None, :]
        out_ref[:, NUM_F + CAT_F + i, :] = e0 + b * (e1 - e0)


def kernel(numeric, categorical, binary, W_num, b_num, bin_emb,
           cat_emb_0, cat_emb_1, cat_emb_2, cat_emb_3, cat_emb_4, cat_emb_5):
    B = numeric.shape[0]
    categorical = categorical.astype(jnp.int32)
    binary = binary.astype(jnp.int32)
    tables = [cat_emb_0, cat_emb_1, cat_emb_2, cat_emb_3, cat_emb_4, cat_emb_5]
    padded = []
    for t in tables:
        v = t.shape[0]
        vp = (v + 7) // 8 * 8
        padded.append(jnp.pad(t, ((0, vp - v), (0, 0))) if vp != v else t)

    grid = (B // BLOCK_B,)
    in_specs = [
        pl.BlockSpec((BLOCK_B, NUM_F), lambda b: (b, 0)),
        pl.BlockSpec((BLOCK_B, CAT_F), lambda b: (b, 0)),
        pl.BlockSpec((BLOCK_B, BIN_F), lambda b: (b, 0)),
        pl.BlockSpec((NUM_F, H), lambda b: (0, 0)),
        pl.BlockSpec((NUM_F, H), lambda b: (0, 0)),
        pl.BlockSpec((BIN_F, 2, H), lambda b: (0, 0, 0)),
    ] + [
        pl.BlockSpec(t.shape, lambda b: (0, 0)) for t in padded
    ]
    out = pl.pallas_call(
        _tc_body,
        grid=grid,
        in_specs=in_specs,
        out_specs=pl.BlockSpec((BLOCK_B, TOKENS, H), lambda b: (b, 0, 0)),
        out_shape=jax.ShapeDtypeStruct((B, TOKENS, H), jnp.float32),
    )(numeric, categorical, binary, W_num, b_num, bin_emb, *padded)
    return out
